# baseline (speedup 1.0000x reference)
import functools

import jax
import jax.numpy as jnp
from jax import lax
from jax.experimental import pallas as pl
from jax.experimental.pallas import tpu as pltpu
from jax.experimental.pallas import tpu_sc as plsc

_SEG = 2048
_HW = 196
_SND_CHUNK = 128


def _make_sc_kernel(N, C, n_seg):
    info = plsc.get_sparse_core_info()
    nw = info.num_cores * info.num_subcores
    halves = nw // n_seg
    rows_w = N // nw
    nk = rows_w // _SND_CHUNK
    ng = C // 16
    mesh = plsc.VectorSubcoreMesh(core_axis_name="c", subcore_axis_name="s")

    @functools.partial(
        pl.kernel,
        out_type=jax.ShapeDtypeStruct((n_seg, halves, C), jnp.float32),
        mesh=mesh,
        scratch_types=[
            pltpu.VMEM((2, _SND_CHUNK, C), jnp.float32),
            pltpu.VMEM((C,), jnp.float32),
            pltpu.SemaphoreType.DMA,
            pltpu.SemaphoreType.DMA,
        ],
    )
    def seg_sums(z_hbm, out_hbm, buf, row_v, sem0, sem1):
        wid = lax.axis_index("s") * info.num_cores + lax.axis_index("c")
        base = wid * rows_w
        sems = (sem0, sem1)

        def copy(k):
            return pltpu.make_async_copy(
                z_hbm.at[pl.ds(base + k * _SND_CHUNK, _SND_CHUNK), :],
                buf.at[k % 2], sems[k % 2])

        copy(0).start()
        accs = tuple(jnp.zeros((16,), jnp.float32) for _ in range(ng))
        for k in range(nk):
            if k + 1 < nk:
                copy(k + 1).start()
            copy(k).wait()
            slot = buf.at[k % 2]

            def body(i, a, slot=slot):
                r = i * 4
                for u in range(4):
                    a = tuple(
                        a[c] + slot[r + u, c * 16:(c + 1) * 16]
                        for c in range(ng))
                return a

            accs = lax.fori_loop(0, _SND_CHUNK // 4, body, accs)
        for c in range(ng):
            row_v[c * 16:(c + 1) * 16] = accs[c]
        pltpu.sync_copy(row_v, out_hbm.at[wid // halves, wid % halves])

    return seg_sums


def _img_body(x_ref, m_ref):
    m = jnp.sum(x_ref[...], axis=(2, 3)) * (1.0 / _HW)
    m_ref[...] = m[:, None, :]


def _mimg_body(mean_ref, mimg_ref):
    m = mean_ref[...][:, 0, :]
    mimg_ref[...] = jnp.broadcast_to(m[None, :, :], mimg_ref.shape)


def _msnd_body(inv_ref, snd_ref, msnd_ref):
    rows = jnp.sum(snd_ref[...], axis=1, keepdims=True) * inv_ref[0]
    msnd_ref[...] = jnp.broadcast_to(rows, msnd_ref.shape)


def kernel(Z_img, Z_snd, snd_splits):
    B, C, H, W = Z_img.shape
    N = Z_snd.shape[0]
    n_seg = N // _SEG

    snd_part = _make_sc_kernel(N, C, n_seg)(Z_snd)

    img_mean = pl.pallas_call(
        _img_body,
        grid=(B,),
        in_specs=[pl.BlockSpec((1, C, H, W), lambda i: (i, 0, 0, 0))],
        out_specs=pl.BlockSpec((1, 1, C), lambda i: (i, 0, 0)),
        out_shape=jax.ShapeDtypeStruct((B, 1, C), jnp.float32),
    )(Z_img)
    M_img = pl.pallas_call(
        _mimg_body,
        grid=(B // 8,),
        in_specs=[pl.BlockSpec((8, 1, C), lambda i: (i, 0, 0))],
        out_specs=pl.BlockSpec((n_seg, 8, C), lambda i: (0, i, 0)),
        out_shape=jax.ShapeDtypeStruct((n_seg, B, C), jnp.float32),
    )(img_mean)

    inv = (1.0 / jnp.asarray(snd_splits).astype(jnp.float32)).reshape(1)
    M_snd = pl.pallas_call(
        _msnd_body,
        grid=(B // 8,),
        in_specs=[
            pl.BlockSpec(memory_space=pltpu.SMEM),
            pl.BlockSpec((n_seg, 2, C), lambda i: (0, 0, 0)),
        ],
        out_specs=pl.BlockSpec((n_seg, 8, C), lambda i: (0, i, 0)),
        out_shape=jax.ShapeDtypeStruct((n_seg, B, C), jnp.float32),
    )(inv, snd_part)
    return (M_img, M_snd)


# speedup vs baseline: 3.3330x; 3.3330x over previous
import functools

import jax
import jax.numpy as jnp
from jax import lax
from jax.experimental import pallas as pl
from jax.experimental.pallas import tpu as pltpu
from jax.experimental.pallas import tpu_sc as plsc

_SEG = 2048
_HW = 196
_SND_CHUNK = 128


def _make_sc_kernel(N, C, n_seg, B):
    info = plsc.get_sparse_core_info()
    nc, ns = info.num_cores, info.num_subcores
    nw = nc * ns
    rows_w = N // nw
    nk = rows_w // _SND_CHUNK
    ng = C // 16
    segs_per_core = n_seg // nc
    mesh = plsc.VectorSubcoreMesh(core_axis_name="c", subcore_axis_name="s")

    @functools.partial(
        pl.kernel,
        out_type=jax.ShapeDtypeStruct((n_seg, B, C), jnp.float32),
        mesh=mesh,
        scratch_types=[
            pltpu.VMEM((2, _SND_CHUNK, C), jnp.float32),
            pltpu.VMEM((1, 1, C), jnp.float32),
            pltpu.VMEM((2, 1, C), jnp.float32),
            pltpu.VMEM((B, C), jnp.float32),
            pltpu.VMEM((16,), jnp.float32),
            pltpu.VMEM_SHARED((ns, 1, C), jnp.float32),
            pltpu.SemaphoreType.DMA,
            pltpu.SemaphoreType.DMA,
            pltpu.SemaphoreType.DMA,
        ],
    )
    def seg_sums(z_hbm, inv_hbm, msnd_hbm, buf, row_v, pair_v, blk_v, inv_v,
                 shared, sem0, sem1, sem2):
        cid = lax.axis_index("c")
        sid = lax.axis_index("s")
        wid = cid * ns + sid
        base = wid * rows_w
        sems = (sem0, sem1)

        def copy(k):
            return pltpu.make_async_copy(
                z_hbm.at[pl.ds(base + k * _SND_CHUNK, _SND_CHUNK), :],
                buf.at[k % 2], sems[k % 2])

        copy(0).start()
        pltpu.async_copy(inv_hbm, inv_v, sem2).wait()
        accs = tuple(jnp.zeros((16,), jnp.float32) for _ in range(ng))
        for k in range(nk):
            if k + 1 < nk:
                copy(k + 1).start()
            copy(k).wait()
            slot = buf.at[k % 2]

            def body(i, a, slot=slot):
                r = i * 4
                for u in range(4):
                    a = tuple(
                        a[c] + slot[r + u, c * 16:(c + 1) * 16]
                        for c in range(ng))
                return a

            accs = lax.fori_loop(0, _SND_CHUNK // 4, body, accs)
        for c in range(ng):
            row_v[0, 0, c * 16:(c + 1) * 16] = accs[c]

        pltpu.sync_copy(row_v, shared.at[pl.ds(sid, 1)])
        plsc.subcore_barrier()

        @pl.when(sid % 2 == 0)
        def _():
            pltpu.sync_copy(shared.at[pl.ds(sid, 2)], pair_v)
            inv = inv_v[0:16]
            for c in range(ng):
                sl = pl.ds(c * 16, 16)
                row_v[0, 0, sl] = (pair_v[0, 0, sl] + pair_v[1, 0, sl]) * inv
            for r in range(B):
                for c in range(ng):
                    sl = pl.ds(c * 16, 16)
                    blk_v[r, sl] = row_v[0, 0, sl]
            seg = cid * segs_per_core + sid // 2
            pltpu.sync_copy(blk_v, msnd_hbm.at[seg])

    return seg_sums


def _img_body(x_ref, mimg_ref):
    m = jnp.sum(x_ref[...], axis=2) * (1.0 / _HW)
    mimg_ref[...] = jnp.broadcast_to(m[None, :, :], mimg_ref.shape)


def kernel(Z_img, Z_snd, snd_splits):
    B, C, H, W = Z_img.shape
    N = Z_snd.shape[0]
    n_seg = N // _SEG

    inv = jnp.full((16,), 1.0, jnp.float32) / jnp.asarray(
        snd_splits).astype(jnp.float32)
    M_snd = _make_sc_kernel(N, C, n_seg, B)(Z_snd, inv)

    Z_img_flat = Z_img.reshape(B, C, H * W)
    M_img = pl.pallas_call(
        _img_body,
        grid=(B // 8,),
        in_specs=[pl.BlockSpec((8, C, H * W), lambda i: (i, 0, 0))],
        out_specs=pl.BlockSpec((n_seg, 8, C), lambda i: (0, i, 0)),
        out_shape=jax.ShapeDtypeStruct((n_seg, B, C), jnp.float32),
    )(Z_img_flat)
    return (M_img, M_snd)
